# baseline (speedup 1.0000x reference)
import functools

import jax
import jax.numpy as jnp
from jax import lax
from jax.experimental import pallas as pl
from jax.experimental.pallas import tpu as pltpu
from jax.experimental.pallas import tpu_sc as plsc

NBINS = 256
START = -15.0
STEP = 30.0 / 256.0
INV_STEP = 256.0 / 30.0
B, N, C = 1024, 100, 4
LANES = 16


def _kde_body(rows_per_w, v_hbm, w_hbm, out_hbm, vv, wv, acc, sem1, sem2):
    nc = 2
    wid = lax.axis_index("s") * nc + lax.axis_index("c")

    v_words = N * rows_per_w
    w_words = C * N * rows_per_w
    acc_words = rows_per_w * NBINS * C

    cp1 = pltpu.async_copy(v_hbm.at[pl.ds(wid * v_words, v_words)], vv, sem1)
    cp2 = pltpu.async_copy(w_hbm.at[pl.ds(wid * w_words, w_words)], wv, sem2)

    zeros = jnp.zeros((LANES,), jnp.float32)

    def zbody(i, _):
        base = i * (8 * LANES)
        for u in range(8):
            acc[pl.ds(base + u * LANES, LANES)] = zeros
        return 0

    lax.fori_loop(0, acc_words // (8 * LANES), zbody, 0)

    cp1.wait()
    cp2.wait()

    iota = lax.iota(jnp.int32, LANES)
    for g in range(rows_per_w // LANES):
        rowbase = (g * LANES + iota) * (NBINS * C)

        def nbody(n, _, g=g, rowbase=rowbase):
            v = vv[pl.ds(n * rows_per_w + g * LANES, LANES)]
            t = (v - START) * INV_STEP
            j = t.astype(jnp.int32)
            e_lo = j.astype(jnp.float32) * STEP + START
            j = j - jnp.where(v < e_lo, 1, 0) + jnp.where(v >= e_lo + STEP, 1, 0)
            j = jnp.minimum(jnp.maximum(j, 0), NBINS - 1)
            base = rowbase + j * C
            for c in range(C):
                w = wv[pl.ds((c * N + n) * rows_per_w + g * LANES, LANES)]
                plsc.addupdate_scatter(acc, [base + c], w)
            return 0

        lax.fori_loop(0, N, nbody, 0)

    pltpu.sync_copy(acc, out_hbm.at[pl.ds(wid * acc_words, acc_words)])


def kernel(value, weights):
    mesh = plsc.VectorSubcoreMesh(core_axis_name="c", subcore_axis_name="s")
    nworkers = mesh.num_cores * mesh.num_subcores
    rows_per_w = B // nworkers

    vW = value.reshape(nworkers, rows_per_w, N).transpose(0, 2, 1).reshape(-1)
    wW = (weights.reshape(nworkers, rows_per_w, N, C)
          .transpose(0, 3, 2, 1).reshape(-1))

    run = pl.kernel(
        functools.partial(_kde_body, rows_per_w),
        out_type=jax.ShapeDtypeStruct((B * NBINS * C,), jnp.float32),
        mesh=mesh,
        compiler_params=pltpu.CompilerParams(needs_layout_passes=False),
        scratch_types=[
            pltpu.VMEM((N * rows_per_w,), jnp.float32),
            pltpu.VMEM((C * N * rows_per_w,), jnp.float32),
            pltpu.VMEM((rows_per_w * NBINS * C,), jnp.float32),
            pltpu.SemaphoreType.DMA,
            pltpu.SemaphoreType.DMA,
        ],
    )
    out = run(vW, wW)
    return out.reshape(B, NBINS, C)


# speedup vs baseline: 5.1844x; 5.1844x over previous
import functools

import jax
import jax.numpy as jnp
from jax import lax
from jax.experimental import pallas as pl
from jax.experimental.pallas import tpu as pltpu
from jax.experimental.pallas import tpu_sc as plsc

NBINS = 256
START = -15.0
STEP = 30.0 / 256.0
INV_STEP = 256.0 / 30.0
B, N, C = 1024, 100, 4
LANES = 16


def _kde_body(rows_per_w, v_hbm, w_hbm, o0, o1, o2, o3,
              vv, wv, a0, a1, a2, a3, sem1, sem2):
    nc = 2
    wid = lax.axis_index("s") * nc + lax.axis_index("c")
    r0 = wid * rows_per_w

    v_words = N * rows_per_w
    w_words = C * N * rows_per_w
    outs = (o0, o1, o2, o3)
    accs = (a0, a1, a2, a3)

    cp1 = pltpu.async_copy(v_hbm.at[pl.ds(wid * v_words, v_words)], vv, sem1)
    cp2 = pltpu.async_copy(w_hbm.at[pl.ds(wid * w_words, w_words)], wv, sem2)

    zeros = jnp.zeros((LANES,), jnp.float32)

    def zbody(r, _):
        for acc in accs:
            for u in range(NBINS // LANES):
                acc[r, pl.ds(u * LANES, LANES)] = zeros
        return 0

    lax.fori_loop(0, rows_per_w, zbody, 0)

    cp1.wait()
    cp2.wait()

    iota = lax.iota(jnp.int32, LANES)
    for g in range(rows_per_w // LANES):
        rows = g * LANES + iota

        def nbody(n, _, g=g, rows=rows):
            v = vv[pl.ds(n * rows_per_w + g * LANES, LANES)]
            t = (v - START) * INV_STEP
            j = t.astype(jnp.int32)
            e_lo = j.astype(jnp.float32) * STEP + START
            j = j - jnp.where(v < e_lo, 1, 0) + jnp.where(v >= e_lo + STEP, 1, 0)
            j = jnp.minimum(jnp.maximum(j, 0), NBINS - 1)
            for c in range(C):
                w = wv[pl.ds((c * N + n) * rows_per_w + g * LANES, LANES)]
                plsc.addupdate_scatter(accs[c], [rows, j], w)
            return 0

        lax.fori_loop(0, N, nbody, 0)

    for c in range(C):
        pltpu.sync_copy(accs[c], outs[c].at[pl.ds(r0, rows_per_w), :])


def kernel(value, weights):
    mesh = plsc.VectorSubcoreMesh(core_axis_name="c", subcore_axis_name="s")
    nworkers = mesh.num_cores * mesh.num_subcores
    rows_per_w = B // nworkers

    vW = value.reshape(nworkers, rows_per_w, N).transpose(0, 2, 1).reshape(-1)
    wW = (weights.reshape(nworkers, rows_per_w, N, C)
          .transpose(0, 3, 2, 1).reshape(-1))

    run = pl.kernel(
        functools.partial(_kde_body, rows_per_w),
        out_type=[jax.ShapeDtypeStruct((B, NBINS), jnp.float32)] * C,
        mesh=mesh,
        compiler_params=pltpu.CompilerParams(needs_layout_passes=False),
        scratch_types=[
            pltpu.VMEM((N * rows_per_w,), jnp.float32),
            pltpu.VMEM((C * N * rows_per_w,), jnp.float32),
        ] + [pltpu.VMEM((rows_per_w, NBINS), jnp.float32)] * C + [
            pltpu.SemaphoreType.DMA,
            pltpu.SemaphoreType.DMA,
        ],
    )
    h0, h1, h2, h3 = run(vW, wW)
    return jnp.stack([h0, h1, h2, h3], axis=2)
